# baseline (speedup 1.0000x reference)
import functools

import jax
import jax.numpy as jnp
from jax import lax
from jax.experimental import pallas as pl
from jax.experimental.pallas import tpu as pltpu
from jax.experimental.pallas import tpu_sc as plsc

_LANES = 16
_NWORKERS = 32
_CHUNK = 16
_QUAD = 4
_LN_EPS = 1e-12

_GATHER_DNUMS = lax.GatherDimensionNumbers(
    offset_dims=(), collapsed_slice_dims=(0,), start_index_map=(0,))


def _lane_shuffle(x, idx):
    return lax.gather(x, idx[:, None], _GATHER_DNUMS, slice_sizes=(1,),
                      mode=lax.GatherScatterMode.PROMISE_IN_BOUNDS)


def _allreduce16(x):
    iota = lax.iota(jnp.int32, _LANES)
    for sh in (8, 4, 2, 1):
        x = x + _lane_shuffle(x, iota ^ sh)
    return x


def _rsqrt16(x):
    i = plsc.bitcast(x, jnp.int32)
    y = plsc.bitcast(jnp.int32(0x5F3759DF) - (i >> 1), jnp.float32)
    half_x = x * jnp.float32(0.5)
    y = y * (jnp.float32(1.5) - half_x * y * y)
    y = y * (jnp.float32(1.5) - half_x * y * y)
    y = y * (jnp.float32(1.5) - half_x * y * y)
    return y


@functools.lru_cache(maxsize=None)
def _build(n_tok, seq_len, hidden):
    spw = n_tok // _NWORKERS
    n_pairs = spw // (2 * _CHUNK)
    nv = hidden // _LANES
    mesh = plsc.VectorSubcoreMesh(core_axis_name="c", subcore_axis_name="s")
    buf_t = pltpu.VMEM((_CHUNK, hidden), jnp.float32)
    vec_t = pltpu.VMEM((hidden,), jnp.float32)

    @functools.partial(
        pl.kernel,
        out_type=jax.ShapeDtypeStruct((n_tok, hidden), jnp.float32),
        mesh=mesh,
        compiler_params=pltpu.CompilerParams(needs_layout_passes=False),
        scratch_types=[
            pltpu.VMEM((spw,), jnp.int32),
            pltpu.VMEM((spw,), jnp.int32),
            pltpu.VMEM((spw,), jnp.int32),
            pltpu.VMEM((spw + _LANES,), jnp.int32),
            buf_t, buf_t,
            buf_t, buf_t,
            pltpu.VMEM((2, hidden), jnp.float32),
            pltpu.VMEM((3, hidden), jnp.float32),
            pltpu.VMEM((6, hidden), jnp.float32),
            vec_t, vec_t,
            pltpu.SemaphoreType.DMA,
            pltpu.SemaphoreType.DMA,
        ],
    )
    def tie_kernel(ids_hbm, tt_hbm, task_hbm, wemb, pemb, temb, kemb,
                   gamma_hbm, beta_hbm, out_hbm,
                   ids_v, tt_v, task_v, cid_v,
                   wa, pa, wb, pb,
                   ttab, ktab, comb, gamma_v, beta_v, sem_g, sem_o):
        wid = lax.axis_index("s") * mesh.num_cores + lax.axis_index("c")
        base = wid * spw
        s_base = lax.rem(base, seq_len)

        pltpu.sync_copy(ids_hbm.at[pl.ds(base, spw)], ids_v)
        pltpu.sync_copy(tt_hbm.at[pl.ds(base, spw)], tt_v)
        pltpu.sync_copy(task_hbm.at[pl.ds(base, spw)], task_v)
        pltpu.sync_copy(gamma_hbm, gamma_v)
        pltpu.sync_copy(beta_hbm, beta_v)
        pltpu.sync_copy(temb, ttab)
        pltpu.sync_copy(kemb, ktab)

        three = jnp.full((_LANES,), 3, jnp.int32)
        zzi = jnp.zeros((_LANES,), jnp.int32)

        def cid_body(i, carry):
            sl = pl.ds(i * _LANES, _LANES)
            cid_v[sl] = tt_v[sl] * three + task_v[sl]
            return carry

        lax.fori_loop(0, spw // _LANES, cid_body, 0)
        cid_v[pl.ds(spw, _LANES)] = zzi

        def comb_body(j, carry):
            sl = pl.ds(j * _LANES, _LANES)
            for r in range(2):
                t_row = ttab[r, sl]
                for kk in range(3):
                    comb[r * 3 + kk, sl] = t_row + ktab[kk, sl]
            return carry

        lax.fori_loop(0, nv, comb_body, 0)

        def issue(c, w, p):
            off = pl.multiple_of(c * _CHUNK, _CHUNK)
            pltpu.async_copy(wemb.at[ids_v[pl.ds(off, _CHUNK)]], w, sem_g)
            pltpu.async_copy(pemb.at[pl.ds(s_base + off, _CHUNK)], p, sem_g)

        def wait_gathers(w):
            for _ in range(2):
                pltpu.make_async_copy(pemb.at[pl.ds(0, _CHUNK)], w,
                                      sem_g).wait()

        def drain_out(w):
            pltpu.make_async_copy(pemb.at[pl.ds(0, _CHUNK)], w, sem_o).wait()

        zz = jnp.zeros((_LANES,), jnp.float32)
        inv_h = jnp.float32(1.0 / hidden)

        def compute(c, w, p):
            off = pl.multiple_of(c * _CHUNK, _CHUNK)

            def quad_body(q, carry):
                t0 = q * _QUAD
                cids = [cid_v[pl.ds(off + t0 + dt, _LANES)][0]
                        for dt in range(_QUAD)]
                acc = [[zz, zz, zz, zz] for _ in range(_QUAD)]
                for j in range(nv):
                    sl = pl.ds(j * _LANES, _LANES)
                    half_j = j & 1
                    for dt in range(_QUAD):
                        tk = t0 + dt
                        v = w[tk, sl] + p[tk, sl] + comb[cids[dt], sl]
                        w[tk, sl] = v
                        acc[dt][half_j] = acc[dt][half_j] + v
                        acc[dt][2 + half_j] = acc[dt][2 + half_j] + v * v
                stats = []
                for dt in range(_QUAD):
                    s = acc[dt][0] + acc[dt][1]
                    ss = acc[dt][2] + acc[dt][3]
                    mean_v = _allreduce16(s) * inv_h
                    var_v = _allreduce16(ss) * inv_h - mean_v * mean_v
                    rstd_v = _rsqrt16(var_v + jnp.float32(_LN_EPS))
                    stats.append((mean_v, rstd_v))
                for j in range(nv):
                    sl = pl.ds(j * _LANES, _LANES)
                    g = gamma_v[sl]
                    b = beta_v[sl]
                    for dt in range(_QUAD):
                        tk = t0 + dt
                        mean_v, rstd_v = stats[dt]
                        a = g * rstd_v
                        w[tk, sl] = (w[tk, sl] - mean_v) * a + b
                return carry

            lax.fori_loop(0, _CHUNK // _QUAD, quad_body, 0)
            pltpu.async_copy(w, out_hbm.at[pl.ds(base + off, _CHUNK)], sem_o)

        issue(0, wa, pa)

        def pair_body(cp, carry):
            c0 = cp * 2
            wait_gathers(wa)

            @pl.when(cp > 0)
            def _():
                drain_out(wb)

            issue(c0 + 1, wb, pb)
            compute(c0, wa, pa)
            wait_gathers(wb)

            @pl.when(cp + 1 < n_pairs)
            def _():
                drain_out(wa)
                issue(c0 + 2, wa, pa)

            compute(c0 + 1, wb, pb)
            return carry

        lax.fori_loop(0, n_pairs, pair_body, 0)
        drain_out(wa)
        drain_out(wb)

    return tie_kernel


def kernel(input_ids, token_type_ids, task_type_ids, word_emb, pos_emb,
           type_emb, task_emb, ln_gamma, ln_beta):
    b, s = input_ids.shape
    hidden = word_emb.shape[1]
    n_tok = b * s
    fn = _build(n_tok, s, hidden)
    out = fn(input_ids.reshape(-1).astype(jnp.int32),
             token_type_ids.reshape(-1).astype(jnp.int32),
             task_type_ids.reshape(-1).astype(jnp.int32),
             word_emb, pos_emb, type_emb, task_emb, ln_gamma, ln_beta)
    return out.reshape(b, s, hidden)


# speedup vs baseline: 1.9240x; 1.9240x over previous
import functools

import jax
import jax.numpy as jnp
from jax import lax
from jax.experimental import pallas as pl
from jax.experimental.pallas import tpu as pltpu
from jax.experimental.pallas import tpu_sc as plsc

_LANES = 16
_NWORKERS = 32
_CHUNK = 16
_QUAD = 2
_LN_EPS = 1e-12

_GATHER_DNUMS = lax.GatherDimensionNumbers(
    offset_dims=(), collapsed_slice_dims=(0,), start_index_map=(0,))


def _lane_shuffle(x, idx):
    return lax.gather(x, idx[:, None], _GATHER_DNUMS, slice_sizes=(1,),
                      mode=lax.GatherScatterMode.PROMISE_IN_BOUNDS)


def _allreduce16(x):
    iota = lax.iota(jnp.int32, _LANES)
    for sh in (8, 4, 2, 1):
        x = x + _lane_shuffle(x, iota ^ sh)
    return x


def _rsqrt16(x):
    i = plsc.bitcast(x, jnp.int32)
    y = plsc.bitcast(jnp.int32(0x5F3759DF) - (i >> 1), jnp.float32)
    half_x = x * jnp.float32(0.5)
    y = y * (jnp.float32(1.5) - half_x * y * y)
    y = y * (jnp.float32(1.5) - half_x * y * y)
    y = y * (jnp.float32(1.5) - half_x * y * y)
    return y


@functools.lru_cache(maxsize=None)
def _build(n_tok, seq_len, hidden):
    spw = n_tok // _NWORKERS
    n_pairs = spw // (2 * _CHUNK)
    nv = hidden // _LANES
    mesh = plsc.VectorSubcoreMesh(core_axis_name="c", subcore_axis_name="s")
    buf_t = pltpu.VMEM((_CHUNK, hidden), jnp.float32)
    vec_t = pltpu.VMEM((hidden,), jnp.float32)

    @functools.partial(
        pl.kernel,
        out_type=jax.ShapeDtypeStruct((n_tok, hidden), jnp.float32),
        mesh=mesh,
        compiler_params=pltpu.CompilerParams(needs_layout_passes=False),
        scratch_types=[
            pltpu.VMEM((spw,), jnp.int32),
            pltpu.VMEM((spw,), jnp.int32),
            pltpu.VMEM((spw,), jnp.int32),
            pltpu.VMEM((spw + _LANES,), jnp.int32),
            buf_t, buf_t,
            buf_t, buf_t,
            pltpu.VMEM((2, hidden), jnp.float32),
            pltpu.VMEM((3, hidden), jnp.float32),
            pltpu.VMEM((6, hidden), jnp.float32),
            vec_t, vec_t,
            pltpu.SemaphoreType.DMA,
            pltpu.SemaphoreType.DMA,
        ],
    )
    def tie_kernel(ids_hbm, tt_hbm, task_hbm, wemb, pemb, temb, kemb,
                   gamma_hbm, beta_hbm, out_hbm,
                   ids_v, tt_v, task_v, cid_v,
                   wa, pa, wb, pb,
                   ttab, ktab, comb, gamma_v, beta_v, sem_g, sem_o):
        wid = lax.axis_index("s") * mesh.num_cores + lax.axis_index("c")
        base = wid * spw
        s_base = lax.rem(base, seq_len)

        pltpu.sync_copy(ids_hbm.at[pl.ds(base, spw)], ids_v)
        pltpu.sync_copy(tt_hbm.at[pl.ds(base, spw)], tt_v)
        pltpu.sync_copy(task_hbm.at[pl.ds(base, spw)], task_v)
        pltpu.sync_copy(gamma_hbm, gamma_v)
        pltpu.sync_copy(beta_hbm, beta_v)
        pltpu.sync_copy(temb, ttab)
        pltpu.sync_copy(kemb, ktab)

        three = jnp.full((_LANES,), 3, jnp.int32)
        zzi = jnp.zeros((_LANES,), jnp.int32)

        def cid_body(i, carry):
            sl = pl.ds(i * _LANES, _LANES)
            cid_v[sl] = tt_v[sl] * three + task_v[sl]
            return carry

        lax.fori_loop(0, spw // _LANES, cid_body, 0)
        cid_v[pl.ds(spw, _LANES)] = zzi

        def comb_body(j, carry):
            sl = pl.ds(j * _LANES, _LANES)
            for r in range(2):
                t_row = ttab[r, sl]
                for kk in range(3):
                    comb[r * 3 + kk, sl] = t_row + ktab[kk, sl]
            return carry

        lax.fori_loop(0, nv, comb_body, 0)

        def issue(c, w, p):
            off = pl.multiple_of(c * _CHUNK, _CHUNK)
            pltpu.async_copy(wemb.at[ids_v[pl.ds(off, _CHUNK)]], w, sem_g)
            pltpu.async_copy(pemb.at[pl.ds(s_base + off, _CHUNK)], p, sem_g)

        def wait_gathers(w):
            for _ in range(2):
                pltpu.make_async_copy(pemb.at[pl.ds(0, _CHUNK)], w,
                                      sem_g).wait()

        def drain_out(w):
            pltpu.make_async_copy(pemb.at[pl.ds(0, _CHUNK)], w, sem_o).wait()

        zz = jnp.zeros((_LANES,), jnp.float32)
        inv_h = jnp.float32(1.0 / hidden)

        def compute(c, w, p):
            off = pl.multiple_of(c * _CHUNK, _CHUNK)

            def quad_body(q, carry):
                t0 = q * _QUAD
                stats = []
                for dt in range(_QUAD):
                    tk = t0 + dt
                    cid = cid_v[pl.ds(off + tk, _LANES)][0]
                    s0 = zz
                    s1 = zz
                    q0 = zz
                    q1 = zz
                    for j in range(nv):
                        sl = pl.ds(j * _LANES, _LANES)
                        v = w[tk, sl] + p[tk, sl] + comb[cid, sl]
                        w[tk, sl] = v
                        if j & 1:
                            s1 = s1 + v
                            q1 = q1 + v * v
                        else:
                            s0 = s0 + v
                            q0 = q0 + v * v
                    mean_v = _allreduce16(s0 + s1) * inv_h
                    var_v = (_allreduce16(q0 + q1) * inv_h
                             - mean_v * mean_v)
                    rstd_v = _rsqrt16(var_v + jnp.float32(_LN_EPS))
                    stats.append((mean_v, rstd_v))
                for j in range(nv):
                    sl = pl.ds(j * _LANES, _LANES)
                    g = gamma_v[sl]
                    b = beta_v[sl]
                    for dt in range(_QUAD):
                        tk = t0 + dt
                        mean_v, rstd_v = stats[dt]
                        a = g * rstd_v
                        w[tk, sl] = (w[tk, sl] - mean_v) * a + b
                return carry

            lax.fori_loop(0, _CHUNK // _QUAD, quad_body, 0)
            pltpu.async_copy(w, out_hbm.at[pl.ds(base + off, _CHUNK)], sem_o)

        issue(0, wa, pa)

        def pair_body(cp, carry):
            c0 = cp * 2
            wait_gathers(wa)

            @pl.when(cp > 0)
            def _():
                drain_out(wb)

            issue(c0 + 1, wb, pb)
            compute(c0, wa, pa)
            wait_gathers(wb)

            @pl.when(cp + 1 < n_pairs)
            def _():
                drain_out(wa)
                issue(c0 + 2, wa, pa)

            compute(c0 + 1, wb, pb)
            return carry

        lax.fori_loop(0, n_pairs, pair_body, 0)
        drain_out(wa)
        drain_out(wb)

    return tie_kernel


def kernel(input_ids, token_type_ids, task_type_ids, word_emb, pos_emb,
           type_emb, task_emb, ln_gamma, ln_beta):
    b, s = input_ids.shape
    hidden = word_emb.shape[1]
    n_tok = b * s
    fn = _build(n_tok, s, hidden)
    out = fn(input_ids.reshape(-1).astype(jnp.int32),
             token_type_ids.reshape(-1).astype(jnp.int32),
             task_type_ids.reshape(-1).astype(jnp.int32),
             word_emb, pos_emb, type_emb, task_emb, ln_gamma, ln_beta)
    return out.reshape(b, s, hidden)


# speedup vs baseline: 5.2570x; 2.7324x over previous
import functools

import jax
import jax.numpy as jnp
from jax import lax
from jax.experimental import pallas as pl
from jax.experimental.pallas import tpu as pltpu
from jax.experimental.pallas import tpu_sc as plsc

_NWORKERS = 32
_CHUNK = 32
_BT = 512
_LN_EPS = 1e-12


@functools.lru_cache(maxsize=None)
def _build_gather(n_tok, hidden):
    spw = n_tok // _NWORKERS
    n_pairs = spw // (2 * _CHUNK)
    mesh = plsc.VectorSubcoreMesh(core_axis_name="c", subcore_axis_name="s")
    buf_t = pltpu.VMEM((_CHUNK, hidden), jnp.float32)

    @functools.partial(
        pl.kernel,
        out_type=jax.ShapeDtypeStruct((n_tok, hidden), jnp.float32),
        mesh=mesh,
        compiler_params=pltpu.CompilerParams(needs_layout_passes=False),
        scratch_types=[
            pltpu.VMEM((spw,), jnp.int32),
            buf_t, buf_t,
            pltpu.SemaphoreType.DMA,
            pltpu.SemaphoreType.DMA,
        ],
    )
    def gather_kernel(ids_hbm, wemb, out_hbm, ids_v, wa, wb, sem_g, sem_o):
        wid = lax.axis_index("s") * mesh.num_cores + lax.axis_index("c")
        base = wid * spw
        pltpu.sync_copy(ids_hbm.at[pl.ds(base, spw)], ids_v)

        def issue(c, w):
            off = pl.multiple_of(c * _CHUNK, _CHUNK)
            for h in range(_CHUNK // 16):
                pltpu.async_copy(wemb.at[ids_v[pl.ds(off + h * 16, 16)]],
                                 w.at[pl.ds(h * 16, 16)], sem_g)

        def wait_gather(w):
            pltpu.make_async_copy(wemb.at[pl.ds(0, _CHUNK)], w, sem_g).wait()

        def writeback(c, w):
            off = pl.multiple_of(c * _CHUNK, _CHUNK)
            pltpu.async_copy(w, out_hbm.at[pl.ds(base + off, _CHUNK)], sem_o)

        def drain_out(w):
            pltpu.make_async_copy(wemb.at[pl.ds(0, _CHUNK)], w, sem_o).wait()

        issue(0, wa)

        def pair_body(cp, carry):
            c0 = cp * 2
            wait_gather(wa)

            @pl.when(cp > 0)
            def _():
                drain_out(wb)
            issue(c0 + 1, wb)
            writeback(c0, wa)
            wait_gather(wb)

            @pl.when(cp + 1 < n_pairs)
            def _():
                drain_out(wa)
                issue(c0 + 2, wa)
            writeback(c0 + 1, wb)
            return carry

        lax.fori_loop(0, n_pairs, pair_body, 0)
        drain_out(wa)
        drain_out(wb)

    return gather_kernel


def _ln_body(wrows_ref, pos_ref, ttf_ref, kkf_ref, temb_ref, kemb_ref,
             gamma_ref, beta_ref, out_ref):
    ttf = ttf_ref[...]
    kkf = kkf_ref[...]
    t0 = temb_ref[0, :]
    t1 = temb_ref[1, :]
    k0 = kemb_ref[0, :]
    k1 = kemb_ref[1, :]
    k2 = kemb_ref[2, :]
    base = wrows_ref[...] + pos_ref[...] + (t0 + k0)[None, :]
    v = (base
         + ttf * (t1 - t0)[None, :]
         + kkf * (k1 - k0)[None, :]
         + (kkf * (kkf - 1.0) * 0.5) * (k2 - 2.0 * k1 + k0)[None, :])
    mean = jnp.mean(v, axis=-1, keepdims=True)
    cv = v - mean
    var = jnp.mean(cv * cv, axis=-1, keepdims=True)
    rstd = lax.rsqrt(var + _LN_EPS)
    out_ref[...] = cv * rstd * gamma_ref[...] + beta_ref[...]


@functools.lru_cache(maxsize=None)
def _build_ln(n_tok, seq_len, hidden):
    n_blocks = n_tok // _BT
    pos_blocks = seq_len // _BT

    return pl.pallas_call(
        _ln_body,
        grid=(n_blocks,),
        in_specs=[
            pl.BlockSpec((_BT, hidden), lambda i: (i, 0)),
            pl.BlockSpec((_BT, hidden), lambda i: (i % pos_blocks, 0)),
            pl.BlockSpec((_BT, 1), lambda i: (i, 0)),
            pl.BlockSpec((_BT, 1), lambda i: (i, 0)),
            pl.BlockSpec((2, hidden), lambda i: (0, 0)),
            pl.BlockSpec((3, hidden), lambda i: (0, 0)),
            pl.BlockSpec((1, hidden), lambda i: (0, 0)),
            pl.BlockSpec((1, hidden), lambda i: (0, 0)),
        ],
        out_specs=pl.BlockSpec((_BT, hidden), lambda i: (i, 0)),
        out_shape=jax.ShapeDtypeStruct((n_tok, hidden), jnp.float32),
    )


def kernel(input_ids, token_type_ids, task_type_ids, word_emb, pos_emb,
           type_emb, task_emb, ln_gamma, ln_beta):
    b, s = input_ids.shape
    hidden = word_emb.shape[1]
    n_tok = b * s
    wrows = _build_gather(n_tok, hidden)(
        input_ids.reshape(-1).astype(jnp.int32), word_emb)
    out = _build_ln(n_tok, s, hidden)(
        wrows,
        pos_emb,
        token_type_ids.reshape(-1, 1).astype(jnp.float32),
        task_type_ids.reshape(-1, 1).astype(jnp.float32),
        type_emb,
        task_emb,
        ln_gamma.reshape(1, -1),
        ln_beta.reshape(1, -1),
    )
    return out.reshape(b, s, hidden)


# speedup vs baseline: 5.3927x; 1.0258x over previous
import functools

import jax
import jax.numpy as jnp
from jax import lax
from jax.experimental import pallas as pl
from jax.experimental.pallas import tpu as pltpu
from jax.experimental.pallas import tpu_sc as plsc

_NWORKERS = 32
_CHUNK = 32
_BT = 1024
_LN_EPS = 1e-12


@functools.lru_cache(maxsize=None)
def _build_gather(n_tok, hidden):
    spw = n_tok // _NWORKERS
    n_pairs = spw // (2 * _CHUNK)
    mesh = plsc.VectorSubcoreMesh(core_axis_name="c", subcore_axis_name="s")
    buf_t = pltpu.VMEM((_CHUNK, hidden), jnp.float32)

    @functools.partial(
        pl.kernel,
        out_type=jax.ShapeDtypeStruct((n_tok, hidden), jnp.float32),
        mesh=mesh,
        compiler_params=pltpu.CompilerParams(needs_layout_passes=False),
        scratch_types=[
            pltpu.VMEM((spw,), jnp.int32),
            buf_t, buf_t,
            pltpu.SemaphoreType.DMA,
            pltpu.SemaphoreType.DMA,
        ],
    )
    def gather_kernel(ids_hbm, wemb, out_hbm, ids_v, wa, wb, sem_g, sem_o):
        wid = lax.axis_index("s") * mesh.num_cores + lax.axis_index("c")
        base = wid * spw
        pltpu.sync_copy(ids_hbm.at[pl.ds(base, spw)], ids_v)

        def issue(c, w):
            off = pl.multiple_of(c * _CHUNK, _CHUNK)
            for h in range(_CHUNK // 16):
                pltpu.async_copy(wemb.at[ids_v[pl.ds(off + h * 16, 16)]],
                                 w.at[pl.ds(h * 16, 16)], sem_g)

        def wait_gather(w):
            pltpu.make_async_copy(wemb.at[pl.ds(0, _CHUNK)], w, sem_g).wait()

        def writeback(c, w):
            off = pl.multiple_of(c * _CHUNK, _CHUNK)
            pltpu.async_copy(w, out_hbm.at[pl.ds(base + off, _CHUNK)], sem_o)

        def drain_out(w):
            pltpu.make_async_copy(wemb.at[pl.ds(0, _CHUNK)], w, sem_o).wait()

        issue(0, wa)

        def pair_body(cp, carry):
            c0 = cp * 2
            wait_gather(wa)

            @pl.when(cp > 0)
            def _():
                drain_out(wb)
            issue(c0 + 1, wb)
            writeback(c0, wa)
            wait_gather(wb)

            @pl.when(cp + 1 < n_pairs)
            def _():
                drain_out(wa)
                issue(c0 + 2, wa)
            writeback(c0 + 1, wb)
            return carry

        lax.fori_loop(0, n_pairs, pair_body, 0)
        drain_out(wa)
        drain_out(wb)

    return gather_kernel


def _ln_body(wrows_ref, pos_ref, ttf_ref, kkf_ref, temb_ref, kemb_ref,
             gamma_ref, beta_ref, out_ref):
    ttf = ttf_ref[...]
    kkf = kkf_ref[...]
    t0 = temb_ref[0, :]
    t1 = temb_ref[1, :]
    k0 = kemb_ref[0, :]
    k1 = kemb_ref[1, :]
    k2 = kemb_ref[2, :]
    base = wrows_ref[...] + pos_ref[...] + (t0 + k0)[None, :]
    v = (base
         + ttf * (t1 - t0)[None, :]
         + kkf * (k1 - k0)[None, :]
         + (kkf * (kkf - 1.0) * 0.5) * (k2 - 2.0 * k1 + k0)[None, :])
    mean = jnp.mean(v, axis=-1, keepdims=True)
    cv = v - mean
    var = jnp.mean(cv * cv, axis=-1, keepdims=True)
    rstd = lax.rsqrt(var + _LN_EPS)
    out_ref[...] = cv * rstd * gamma_ref[...] + beta_ref[...]


@functools.lru_cache(maxsize=None)
def _build_ln(n_tok, seq_len, hidden):
    n_blocks = n_tok // _BT
    pos_blocks = seq_len // _BT

    return pl.pallas_call(
        _ln_body,
        grid=(n_blocks,),
        in_specs=[
            pl.BlockSpec((_BT, hidden), lambda i: (i, 0)),
            pl.BlockSpec((_BT, hidden), lambda i: (i % pos_blocks, 0)),
            pl.BlockSpec((_BT, 1), lambda i: (i, 0)),
            pl.BlockSpec((_BT, 1), lambda i: (i, 0)),
            pl.BlockSpec((2, hidden), lambda i: (0, 0)),
            pl.BlockSpec((3, hidden), lambda i: (0, 0)),
            pl.BlockSpec((1, hidden), lambda i: (0, 0)),
            pl.BlockSpec((1, hidden), lambda i: (0, 0)),
        ],
        out_specs=pl.BlockSpec((_BT, hidden), lambda i: (i, 0)),
        out_shape=jax.ShapeDtypeStruct((n_tok, hidden), jnp.float32),
    )


def kernel(input_ids, token_type_ids, task_type_ids, word_emb, pos_emb,
           type_emb, task_emb, ln_gamma, ln_beta):
    b, s = input_ids.shape
    hidden = word_emb.shape[1]
    n_tok = b * s
    wrows = _build_gather(n_tok, hidden)(
        input_ids.reshape(-1).astype(jnp.int32), word_emb)
    out = _build_ln(n_tok, s, hidden)(
        wrows,
        pos_emb,
        token_type_ids.reshape(-1, 1).astype(jnp.float32),
        task_type_ids.reshape(-1, 1).astype(jnp.float32),
        type_emb,
        task_emb,
        ln_gamma.reshape(1, -1),
        ln_beta.reshape(1, -1),
    )
    return out.reshape(b, s, hidden)


# speedup vs baseline: 5.7195x; 1.0606x over previous
import functools

import jax
import jax.numpy as jnp
from jax import lax
from jax.experimental import pallas as pl
from jax.experimental.pallas import tpu as pltpu
from jax.experimental.pallas import tpu_sc as plsc

_NWORKERS = 32
_CHUNK = 32
_BT = 1024
_LN_EPS = 1e-12


@functools.lru_cache(maxsize=None)
def _build_gather(n_tok, hidden):
    spw = n_tok // _NWORKERS
    n_pairs = spw // (2 * _CHUNK)
    mesh = plsc.VectorSubcoreMesh(core_axis_name="c", subcore_axis_name="s")
    buf_t = pltpu.VMEM((_CHUNK, hidden), jnp.float32)

    @functools.partial(
        pl.kernel,
        out_type=jax.ShapeDtypeStruct((n_tok, hidden), jnp.float32),
        mesh=mesh,
        compiler_params=pltpu.CompilerParams(needs_layout_passes=False),
        scratch_types=[
            pltpu.VMEM((spw,), jnp.int32),
            buf_t, buf_t,
            pltpu.SemaphoreType.DMA,
            pltpu.SemaphoreType.DMA,
        ],
    )
    def gather_kernel(ids_hbm, wemb, out_hbm, ids_v, wa, wb, sem_g, sem_o):
        wid = lax.axis_index("s") * mesh.num_cores + lax.axis_index("c")
        base = wid * spw
        pltpu.sync_copy(ids_hbm.at[pl.ds(base, spw)], ids_v)

        def issue(c, w):
            off = pl.multiple_of(c * _CHUNK, _CHUNK)
            for h in range(_CHUNK // 16):
                pltpu.async_copy(wemb.at[ids_v[pl.ds(off + h * 16, 16)]],
                                 w.at[pl.ds(h * 16, 16)], sem_g)

        def wait_gather(w):
            pltpu.make_async_copy(wemb.at[pl.ds(0, _CHUNK)], w, sem_g).wait()

        def writeback(c, w):
            off = pl.multiple_of(c * _CHUNK, _CHUNK)
            pltpu.async_copy(w, out_hbm.at[pl.ds(base + off, _CHUNK)], sem_o)

        def drain_out(w):
            pltpu.make_async_copy(wemb.at[pl.ds(0, _CHUNK)], w, sem_o).wait()

        issue(0, wa)

        def pair_body(cp, carry):
            c0 = cp * 2
            wait_gather(wa)

            @pl.when(cp > 0)
            def _():
                drain_out(wb)
            issue(c0 + 1, wb)
            writeback(c0, wa)
            wait_gather(wb)

            @pl.when(cp + 1 < n_pairs)
            def _():
                drain_out(wa)
                issue(c0 + 2, wa)
            writeback(c0 + 1, wb)
            return carry

        lax.fori_loop(0, n_pairs, pair_body, 0)
        drain_out(wa)
        drain_out(wb)

    return gather_kernel


def _ln_body(wrows_ref, pos_ref, ttf_ref, kkf_ref, temb_ref, kemb_ref,
             gamma_ref, beta_ref, out_ref):
    ttf = ttf_ref[...]
    kkf = kkf_ref[...]
    t0 = temb_ref[0, :]
    t1 = temb_ref[1, :]
    k0 = kemb_ref[0, :]
    k1 = kemb_ref[1, :]
    k2 = kemb_ref[2, :]
    base = wrows_ref[...] + pos_ref[...] + (t0 + k0)[None, :]
    v = (base
         + ttf * (t1 - t0)[None, :]
         + kkf * (k1 - k0)[None, :]
         + (kkf * (kkf - 1.0) * 0.5) * (k2 - 2.0 * k1 + k0)[None, :])
    mean = jnp.mean(v, axis=-1, keepdims=True)
    cv = v - mean
    var = jnp.mean(cv * cv, axis=-1, keepdims=True)
    rstd = lax.rsqrt(var + _LN_EPS)
    out_ref[...] = cv * rstd * gamma_ref[...] + beta_ref[...]


@functools.lru_cache(maxsize=None)
def _build_ln(n_tok, seq_len, hidden):
    s_blocks = seq_len // _BT
    n_batch = n_tok // seq_len

    def tok(si, bi):
        return (bi * s_blocks + si, 0)

    return pl.pallas_call(
        _ln_body,
        grid=(s_blocks, n_batch),
        in_specs=[
            pl.BlockSpec((_BT, hidden), tok),
            pl.BlockSpec((_BT, hidden), lambda si, bi: (si, 0)),
            pl.BlockSpec((_BT, 1), tok),
            pl.BlockSpec((_BT, 1), tok),
            pl.BlockSpec((2, hidden), lambda si, bi: (0, 0)),
            pl.BlockSpec((3, hidden), lambda si, bi: (0, 0)),
            pl.BlockSpec((1, hidden), lambda si, bi: (0, 0)),
            pl.BlockSpec((1, hidden), lambda si, bi: (0, 0)),
        ],
        out_specs=pl.BlockSpec((_BT, hidden), tok),
        out_shape=jax.ShapeDtypeStruct((n_tok, hidden), jnp.float32),
    )


def kernel(input_ids, token_type_ids, task_type_ids, word_emb, pos_emb,
           type_emb, task_emb, ln_gamma, ln_beta):
    b, s = input_ids.shape
    hidden = word_emb.shape[1]
    n_tok = b * s
    wrows = _build_gather(n_tok, hidden)(
        input_ids.reshape(-1).astype(jnp.int32), word_emb)
    out = _build_ln(n_tok, s, hidden)(
        wrows,
        pos_emb,
        token_type_ids.reshape(-1, 1).astype(jnp.float32),
        task_type_ids.reshape(-1, 1).astype(jnp.float32),
        type_emb,
        task_emb,
        ln_gamma.reshape(1, -1),
        ln_beta.reshape(1, -1),
    )
    return out.reshape(b, s, hidden)
